# Initial kernel scaffold; baseline (speedup 1.0000x reference)
import jax
import jax.numpy as jnp
from jax.experimental import pallas as pl


def kernel(x, edge_index, edge_attr, atom1, atom2, bond1, bond2, W1, b1, W2, b2, gamma, beta):
    raise NotImplementedError("write your pallas kernel here")


# speedup vs baseline: 1.6604x; 1.6604x over previous
import functools

import jax
import jax.numpy as jnp
from jax import lax
from jax.experimental import pallas as pl
from jax.experimental.pallas import tpu as pltpu
from jax.experimental.pallas import tpu_sc as plsc

N = 10000
E = 320000
EMB = 128
NUM_LAYER = 5
NUM_ATOM_TYPE = 120
NUM_CHIRALITY = 3
NUM_BOND_TYPE = 7
NUM_BOND_DIR = 4

NC, NS = 2, 16
NW = NC * NS
CHUNK = 128
E_TOT = E + N
EPT = 10368
NCH = EPT // CHUNK
E_PAD = NW * EPT
ACC_ROWS = 10112
RPT = ACC_ROWS // NS
JUNK_BASE = 10008
N_JUNK = 104


@functools.lru_cache(maxsize=None)
def _sc_kernel():
    mesh = plsc.VectorSubcoreMesh(core_axis_name="c", subcore_axis_name="s",
                                  num_cores=NC, num_subcores=NS)

    @functools.partial(
        pl.kernel,
        out_type=jax.ShapeDtypeStruct((NC, ACC_ROWS, EMB), jnp.float32),
        mesh=mesh,
        scratch_types=[
            pltpu.VMEM_SHARED((ACC_ROWS, EMB), jnp.float32),
            pltpu.VMEM((CHUNK,), jnp.int32),
            pltpu.VMEM((CHUNK,), jnp.int32),
            pltpu.VMEM((NCH, CHUNK), jnp.int32),
            pltpu.VMEM((CHUNK, EMB), jnp.float32),
            pltpu.VMEM((CHUNK, EMB), jnp.float32),
        ],
    )
    def sc_gather_scatter(h, bc, srcs, rids, dsts, zeros, parts, acc, srcc,
                          ridc, dstv, buf, buf2):
        c = lax.axis_index("c")
        s = lax.axis_index("s")
        wid = s * NC + c
        pltpu.sync_copy(zeros.at[pl.ds(s * RPT, RPT)],
                        acc.at[pl.ds(s * RPT, RPT)])
        pltpu.sync_copy(dsts.at[wid], dstv)
        plsc.subcore_barrier()

        def chunk_body(j, _):
            pltpu.sync_copy(srcs.at[wid, pl.ds(j * CHUNK, CHUNK)], srcc)
            pltpu.sync_copy(rids.at[wid, pl.ds(j * CHUNK, CHUNK)], ridc)
            pltpu.sync_copy(h.at[srcc], buf)
            pltpu.sync_copy(bc.at[ridc], buf2)

            def add_row(r, _):
                for k in range(EMB // 16):
                    buf[r, pl.ds(k * 16, 16)] = (buf[r, pl.ds(k * 16, 16)]
                                                 + buf2[r, pl.ds(k * 16, 16)])
                return 0

            lax.fori_loop(0, CHUNK, add_row, 0)
            pltpu.sync_copy(buf, acc.at[dstv.at[j]], add=True)
            return 0

        lax.fori_loop(0, NCH, chunk_body, 0)
        plsc.subcore_barrier()
        pltpu.sync_copy(acc.at[pl.ds(s * RPT, RPT)],
                        parts.at[c, pl.ds(s * RPT, RPT)])

    return sc_gather_scatter


def _encode_body(x_ref, a1_ref, a2_ref, h_ref):
    x0 = jnp.clip(x_ref[:, 0:1], 0, NUM_ATOM_TYPE - 1)
    x1 = jnp.clip(x_ref[:, 1:2], 0, NUM_CHIRALITY - 1)
    i1 = lax.broadcasted_iota(jnp.int32, (N, NUM_ATOM_TYPE), 1)
    i2 = lax.broadcasted_iota(jnp.int32, (N, NUM_CHIRALITY), 1)
    oh1 = (i1 == x0).astype(jnp.float32)
    oh2 = (i2 == x1).astype(jnp.float32)
    h_ref[...] = (jnp.dot(oh1, a1_ref[...], preferred_element_type=jnp.float32,
                          precision=lax.Precision.HIGHEST)
                  + jnp.dot(oh2, a2_ref[...], preferred_element_type=jnp.float32,
                            precision=lax.Precision.HIGHEST))


_encode = pl.pallas_call(
    _encode_body,
    out_shape=jax.ShapeDtypeStruct((N, EMB), jnp.float32),
    compiler_params=pltpu.CompilerParams(vmem_limit_bytes=100 * 1024 * 1024),
)


def _mlp_body(parts_ref, w1_ref, b1_ref, w2_ref, out_ref):
    aggr = parts_ref[0, :N, :] + parts_ref[1, :N, :]
    z = jnp.maximum(
        jnp.dot(aggr, w1_ref[...], preferred_element_type=jnp.float32)
        + b1_ref[...], 0.0)
    out_ref[...] = jnp.dot(z, w2_ref[...], preferred_element_type=jnp.float32)


_mlp = pl.pallas_call(
    _mlp_body, out_shape=jax.ShapeDtypeStruct((N, EMB), jnp.float32))


def _norm_body(h_ref, m_ref, v_ref, g_ref, be_ref, out_ref, *, last):
    hh = ((h_ref[...] - m_ref[...]) / jnp.sqrt(v_ref[...] + 1e-5)
          * g_ref[...] + be_ref[...])
    if not last:
        hh = jnp.maximum(hh, 0.0)
    out_ref[...] = hh


_norm = [
    pl.pallas_call(
        functools.partial(_norm_body, last=(l == NUM_LAYER - 1)),
        out_shape=jax.ShapeDtypeStruct((N, EMB), jnp.float32),
    )
    for l in range(NUM_LAYER)
]


def kernel(x, edge_index, edge_attr, atom1, atom2, bond1, bond2, W1, b1, W2,
           b2, gamma, beta):
    i32 = jnp.int32
    src = edge_index[0].astype(i32)
    dst = edge_index[1].astype(i32)
    ea = edge_attr.astype(i32)
    loop = jnp.arange(N, dtype=i32)
    src_f = jnp.concatenate([src, loop])
    dst_f = jnp.concatenate([dst, loop])
    rid_f = jnp.concatenate([ea[:, 0] * 4 + ea[:, 1],
                             jnp.full((N,), 27, i32)])
    order = jnp.argsort(dst_f, stable=True)
    ss = src_f[order]
    ds = dst_f[order]
    rs = rid_f[order]
    pad = E_PAD - E_TOT
    ss = jnp.concatenate([ss, (jnp.arange(pad, dtype=i32) * 97) % N])
    rs = jnp.concatenate([rs, jnp.full((pad,), 27, i32)])
    ds = jnp.concatenate([ds, JUNK_BASE + (jnp.arange(pad, dtype=i32) % N_JUNK)])
    srcs = ss.reshape(NW, EPT)
    rids = rs.reshape(NW, EPT)
    dsts = ds.reshape(NW, NCH, CHUNK)

    t0 = jnp.minimum(jnp.arange(32) // 4, NUM_BOND_TYPE - 1)
    t1 = jnp.arange(32) % 4
    bc_all = bond1[:, t0, :] + bond2[:, t1, :]

    zeros_h = jnp.zeros((ACC_ROWS, EMB), jnp.float32)

    sc_gather_scatter = _sc_kernel()
    h = _encode(x.astype(i32), atom1, atom2)
    for l in range(NUM_LAYER):
        parts = sc_gather_scatter(h, bc_all[l], srcs, rids, dsts, zeros_h)
        hh = _mlp(parts, W1[l], b1[l], W2[l]) + b2[l]
        mean = jnp.mean(hh, axis=0)
        var = jnp.var(hh, axis=0)
        h = _norm[l](hh, mean.reshape(1, EMB), var.reshape(1, EMB),
                     gamma[l].reshape(1, EMB), beta[l].reshape(1, EMB))
    return h
